# baseline (speedup 1.0000x reference)
import jax
import jax.numpy as jnp
from jax.experimental import pallas as pl
from jax.experimental.pallas import tpu as pltpu

_NCHUNK = 16


def _copy_body(w_ref, o_ref, sems):
    rows = w_ref.shape[0]
    chunk = rows // _NCHUNK
    copies = [
        pltpu.make_async_copy(
            w_ref.at[pl.ds(i * chunk, chunk)],
            o_ref.at[pl.ds(i * chunk, chunk)],
            sems.at[i],
        )
        for i in range(_NCHUNK)
    ]
    for c in copies:
        c.start()
    for c in copies:
        c.wait()


def kernel(x, W_pos):
    seq_len, d_model = W_pos.shape
    return pl.pallas_call(
        _copy_body,
        in_specs=[pl.BlockSpec(memory_space=pltpu.MemorySpace.HBM)],
        out_specs=pl.BlockSpec(memory_space=pltpu.MemorySpace.HBM),
        out_shape=jax.ShapeDtypeStruct((seq_len, d_model), W_pos.dtype),
        scratch_shapes=[pltpu.SemaphoreType.DMA((_NCHUNK,))],
    )(W_pos)


# speedup vs baseline: 47.0896x; 47.0896x over previous
import jax
import jax.numpy as jnp
from jax.experimental import pallas as pl
from jax.experimental.pallas import tpu as pltpu

_BLOCK_ROWS = 512


def _copy_body(w_ref, o_ref):
    o_ref[...] = w_ref[...]


def kernel(x, W_pos):
    seq_len, d_model = W_pos.shape
    grid = seq_len // _BLOCK_ROWS
    return pl.pallas_call(
        _copy_body,
        grid=(grid,),
        in_specs=[pl.BlockSpec((_BLOCK_ROWS, d_model), lambda i: (i, 0))],
        out_specs=pl.BlockSpec((_BLOCK_ROWS, d_model), lambda i: (i, 0)),
        out_shape=jax.ShapeDtypeStruct((seq_len, d_model), W_pos.dtype),
    )(W_pos)
